# baseline (speedup 1.0000x reference)
import functools

import jax
import jax.numpy as jnp
from jax import lax
from jax.experimental import pallas as pl
from jax.experimental.pallas import tpu as pltpu
from jax.experimental.pallas import tpu_sc as plsc

N = 10000
E = 320000
EH = E // 2
D = 128
DEPTH = 4

NC, NS = 2, 16
NW = NC * NS
GC = 80
EPW = E // NW
CPW = EPW // GC
NB = 5
GR = NB * GC
NG = EPW // GR
ZC = GC
NZC = N // ZC

BM = 8000
BN = 5000

_MESH = plsc.VectorSubcoreMesh(core_axis_name="c", subcore_axis_name="s")


def _mm_body(a_ref, w_ref, o_ref):
    o_ref[...] = jnp.dot(a_ref[...].astype(w_ref.dtype), w_ref[...],
                         preferred_element_type=jnp.float32
                         ).astype(o_ref.dtype)


def _matmul(a, w, bm, out_dtype=jnp.float32):
    rows, k = a.shape
    return pl.pallas_call(
        _mm_body,
        grid=(rows // bm,),
        in_specs=[pl.BlockSpec((bm, k), lambda i: (i, 0)),
                  pl.BlockSpec(w.shape, lambda i: (0, 0))],
        out_specs=pl.BlockSpec((bm, w.shape[1]), lambda i: (i, 0)),
        out_shape=jax.ShapeDtypeStruct((rows, w.shape[1]), out_dtype),
    )(a, w)


def _estep0_body(g_ref, a_ref, we_ref, w_ref, msg_ref, hnew_ref):
    ea = jnp.dot(a_ref[...], we_ref[...], preferred_element_type=jnp.float32)
    m = jnp.maximum(g_ref[...] + ea, 0.0)
    msg_ref[...] = m
    hnew_ref[...] = jnp.dot(m.astype(jnp.bfloat16), w_ref[...],
                            preferred_element_type=jnp.float32
                            ).astype(jnp.bfloat16)


def _estep0(gb, eah, we, whb):
    nb, hb = E // BM, EH // BM
    return pl.pallas_call(
        _estep0_body,
        grid=(nb,),
        in_specs=[pl.BlockSpec((BM, D), lambda i: (i, 0)),
                  pl.BlockSpec((BM, 16), lambda i: (i % hb, 0)),
                  pl.BlockSpec((16, D), lambda i: (0, 0)),
                  pl.BlockSpec((D, D), lambda i: (0, 0))],
        out_specs=[pl.BlockSpec((BM, D), lambda i: (i, 0)),
                   pl.BlockSpec((BM, D), lambda i: (i, 0))],
        out_shape=[jax.ShapeDtypeStruct((E, D), jnp.float32),
                   jax.ShapeDtypeStruct((E, D), jnp.bfloat16)],
    )(gb, eah, we, whb)


def _estep_body(g_ref, a_ref, we_ref, h_ref, w_ref, msg_ref, hnew_ref):
    ea = jnp.dot(a_ref[...], we_ref[...], preferred_element_type=jnp.float32)
    m = jnp.maximum(g_ref[...] + ea
                    - h_ref[...].astype(jnp.float32), 0.0)
    msg_ref[...] = m
    hnew_ref[...] = jnp.dot(m.astype(jnp.bfloat16), w_ref[...],
                            preferred_element_type=jnp.float32
                            ).astype(jnp.bfloat16)


def _estep(gb, eah, we, hmat, whb):
    nb, hb = E // BM, EH // BM
    return pl.pallas_call(
        _estep_body,
        grid=(nb,),
        in_specs=[pl.BlockSpec((BM, D), lambda i: (i, 0)),
                  pl.BlockSpec((BM, 16), lambda i: (i % hb, 0)),
                  pl.BlockSpec((16, D), lambda i: (0, 0)),
                  pl.BlockSpec((BM, D), lambda i: ((i + hb) % nb, 0)),
                  pl.BlockSpec((D, D), lambda i: (0, 0))],
        out_specs=[pl.BlockSpec((BM, D), lambda i: (i, 0)),
                   pl.BlockSpec((BM, D), lambda i: (i, 0))],
        out_shape=[jax.ShapeDtypeStruct((E, D), jnp.float32),
                   jax.ShapeDtypeStruct((E, D), jnp.bfloat16)],
    )(gb, eah, we, hmat, whb)


def _edge_body(g_ref, a_ref, we_ref, h_ref, o_ref):
    ea = jnp.dot(a_ref[...], we_ref[...], preferred_element_type=jnp.float32)
    o_ref[...] = jnp.maximum(g_ref[...] + ea
                             - h_ref[...].astype(jnp.float32), 0.0)


def _edge_combine(gb, eah, we, hmat):
    nb, hb = E // BM, EH // BM
    return pl.pallas_call(
        _edge_body,
        grid=(nb,),
        in_specs=[pl.BlockSpec((BM, D), lambda i: (i, 0)),
                  pl.BlockSpec((BM, 16), lambda i: (i % hb, 0)),
                  pl.BlockSpec((16, D), lambda i: (0, 0)),
                  pl.BlockSpec((BM, D), lambda i: ((i + hb) % nb, 0))],
        out_specs=pl.BlockSpec((BM, D), lambda i: (i, 0)),
        out_shape=jax.ShapeDtypeStruct((E, D), jnp.float32),
    )(gb, eah, we, hmat)


def _tcomb_body(xh_ref, p0_ref, p1_ref, w_ref, o_ref):
    s = p0_ref[...] + p1_ref[...]
    o_ref[...] = xh_ref[...] + jnp.dot(s, w_ref[...],
                                       preferred_element_type=jnp.float32)


def _tcombine(xh, p0, p1, w):
    return pl.pallas_call(
        _tcomb_body,
        grid=(N // BN,),
        in_specs=[pl.BlockSpec((BN, D), lambda i: (i, 0)),
                  pl.BlockSpec((BN, D), lambda i: (i, 0)),
                  pl.BlockSpec((BN, D), lambda i: (i, 0)),
                  pl.BlockSpec((D, D), lambda i: (0, 0))],
        out_specs=pl.BlockSpec((BN, D), lambda i: (i, 0)),
        out_shape=jax.ShapeDtypeStruct((N, D), jnp.float32),
    )(xh, p0, p1, w)


def _final_body(x_ref, p0_ref, p1_ref, w1_ref, w2_ref, b_ref, o_ref):
    i = pl.program_id(0)
    m = p0_ref[...] + p1_ref[...]
    h = jnp.dot(x_ref[...], w1_ref[...], preferred_element_type=jnp.float32)
    h = h + jnp.dot(m, w2_ref[...], preferred_element_type=jnp.float32)
    h = jnp.maximum(h + b_ref[...], 0.0)
    s = jnp.sum(h, axis=0, keepdims=True) * (1.0 / N)

    @pl.when(i == 0)
    def _():
        o_ref[...] = jnp.zeros_like(o_ref)

    o_ref[...] += s


def _final(x, p0, p1, w1, w2, b):
    return pl.pallas_call(
        _final_body,
        grid=(N // BN,),
        in_specs=[pl.BlockSpec((BN, D), lambda i: (i, 0)),
                  pl.BlockSpec((BN, D), lambda i: (i, 0)),
                  pl.BlockSpec((BN, D), lambda i: (i, 0)),
                  pl.BlockSpec((D, D), lambda i: (0, 0)),
                  pl.BlockSpec((D, D), lambda i: (0, 0)),
                  pl.BlockSpec((1, D), lambda i: (0, 0))],
        out_specs=pl.BlockSpec((1, D), lambda i: (0, 0)),
        out_shape=jax.ShapeDtypeStruct((1, D), jnp.float32),
    )(x, p0, p1, w1, w2, b)


def _gather_sc_body(tab_ref, idx_hbm, out_ref, idx_v, buf0, buf1,
                    gsem0, gsem1, wsem0, wsem1):
    c = lax.axis_index("c")
    s = lax.axis_index("s")
    wid = c * NS + s
    e0 = wid * EPW
    pltpu.sync_copy(idx_hbm.at[wid], idx_v)

    bufs = (buf0, buf1)
    gsems = (gsem0, gsem1)
    wsems = (wsem0, wsem1)
    wdesc = [None, None]
    for g in range(NG):
        b = g & 1
        if wdesc[b] is not None:
            wdesc[b].wait()
        gds = [
            pltpu.async_copy(tab_ref.at[idx_v.at[g * NB + j]],
                             bufs[b].at[pl.ds(j * GC, GC)], gsems[b])
            for j in range(NB)
        ]
        for dsc in gds:
            dsc.wait()
        wdesc[b] = pltpu.async_copy(
            bufs[b], out_ref.at[pl.ds(e0 + g * GR, GR)], wsems[b])
    wdesc[0].wait()
    wdesc[1].wait()


@functools.partial(
    pl.kernel,
    out_type=jax.ShapeDtypeStruct((E, D), jnp.float32),
    mesh=_MESH,
    scratch_types=[
        pltpu.VMEM((CPW, GC), jnp.int32),
        pltpu.VMEM((GR, D), jnp.float32),
        pltpu.VMEM((GR, D), jnp.float32),
        pltpu.SemaphoreType.DMA,
        pltpu.SemaphoreType.DMA,
        pltpu.SemaphoreType.DMA,
        pltpu.SemaphoreType.DMA,
    ],
)
def _gather(tab_ref, idx_hbm, out_ref, idx_v, buf0, buf1,
            gsem0, gsem1, wsem0, wsem1):
    _gather_sc_body(tab_ref, idx_hbm, out_ref, idx_v, buf0, buf1,
                    gsem0, gsem1, wsem0, wsem1)


def _segsum_sc_body(msg_ref, idx_hbm, out_ref, idx_v, mb0, mb1, mb2, acc,
                    lsem0, lsem1, lsem2, ssem0, ssem1, ssem2):
    c = lax.axis_index("c")
    s = lax.axis_index("s")
    wid = c * NS + s
    e0 = wid * EPW
    pltpu.sync_copy(idx_hbm.at[wid], idx_v)

    zero = jnp.zeros((16,), jnp.float32)

    def zrow(i, _):
        for j in range(D // 16):
            mb0[i, pl.ds(j * 16, 16)] = zero
        return 0

    lax.fori_loop(0, ZC, zrow, 0)
    for z in range(NZC // NS + 1):
        ci = s + NS * z

        @pl.when(ci < NZC)
        def _():
            pltpu.sync_copy(mb0, acc.at[pl.ds(ci * ZC, ZC)])

    plsc.subcore_barrier()

    mbs = (mb0, mb1, mb2)
    lsems = (lsem0, lsem1, lsem2)
    ssems = (ssem0, ssem1, ssem2)
    ldesc = [None, None, None]
    sdesc = [None, None, None]
    for k in range(CPW + 1):
        if k < CPW:
            b = k % 3
            if sdesc[b] is not None:
                sdesc[b].wait()
            ldesc[b] = pltpu.async_copy(
                msg_ref.at[pl.ds(e0 + k * GC, GC)], mbs[b], lsems[b])
        if k > 0:
            pb = (k - 1) % 3
            ldesc[pb].wait()
            sdesc[pb] = pltpu.async_copy(
                mbs[pb], acc.at[idx_v.at[k - 1]], ssems[pb], add=True)
    for b in range(3):
        if sdesc[b] is not None:
            sdesc[b].wait()
    plsc.subcore_barrier()
    for z in range(NZC // NS + 1):
        ci = s + NS * z

        @pl.when(ci < NZC)
        def _():
            pltpu.sync_copy(acc.at[pl.ds(ci * ZC, ZC)],
                            out_ref.at[c, pl.ds(ci * ZC, ZC)])


@functools.partial(
    pl.kernel,
    out_type=jax.ShapeDtypeStruct((NC, N, D), jnp.float32),
    mesh=_MESH,
    scratch_types=[
        pltpu.VMEM((CPW, GC), jnp.int32),
        pltpu.VMEM((GC, D), jnp.float32),
        pltpu.VMEM((GC, D), jnp.float32),
        pltpu.VMEM((GC, D), jnp.float32),
        pltpu.VMEM_SHARED((N, D), jnp.float32),
        pltpu.SemaphoreType.DMA,
        pltpu.SemaphoreType.DMA,
        pltpu.SemaphoreType.DMA,
        pltpu.SemaphoreType.DMA,
        pltpu.SemaphoreType.DMA,
        pltpu.SemaphoreType.DMA,
    ],
)
def _segsum(msg_ref, idx_hbm, out_ref, idx_v, mb0, mb1, mb2, acc,
            lsem0, lsem1, lsem2, ssem0, ssem1, ssem2):
    _segsum_sc_body(msg_ref, idx_hbm, out_ref, idx_v, mb0, mb1, mb2, acc,
                    lsem0, lsem1, lsem2, ssem0, ssem1, ssem2)


def kernel(x, edge_attr, W_msg_i, W_h, W_o, b_o, edge_index):
    src = edge_index[0].astype(jnp.int32)
    dst = edge_index[1].astype(jnp.int32)
    srcr = src.reshape(NW, CPW, GC)
    dstr = dst.reshape(NW, CPW, GC)
    Wx = W_msg_i[:D]
    We = W_msg_i[D:]

    xh = _matmul(x, Wx, BN)
    eah = edge_attr[:EH]
    whb = W_h.astype(jnp.bfloat16)

    g0 = _gather(xh, srcr)
    msg, hmat = _estep0(g0, eah, We, whb)

    for _ in range(DEPTH - 2):
        parts = _segsum(msg, dstr)
        tb = _tcombine(xh, parts[0], parts[1], W_h)
        gb = _gather(tb, srcr)
        msg, hmat = _estep(gb, eah, We, hmat, whb)

    parts = _segsum(msg, dstr)
    tb = _tcombine(xh, parts[0], parts[1], W_h)
    gb = _gather(tb, srcr)
    msg = _edge_combine(gb, eah, We, hmat)

    parts = _segsum(msg, dstr)
    return _final(x, parts[0], parts[1], W_o[:D], W_o[D:],
                  b_o.reshape(1, D))


# speedup vs baseline: 1.0375x; 1.0375x over previous
import functools

import jax
import jax.numpy as jnp
from jax import lax
from jax.experimental import pallas as pl
from jax.experimental.pallas import tpu as pltpu
from jax.experimental.pallas import tpu_sc as plsc

N = 10000
E = 320000
EH = E // 2
D = 128
DEPTH = 4

NC, NS = 2, 16
NW = NC * NS
GC = 80
EPW = E // NW
CPW = EPW // GC
NB = 5
GR = NB * GC
NG = EPW // GR
ZC = GC
NZC = N // ZC

BM = 8000
BN = 5000

_MESH = plsc.VectorSubcoreMesh(core_axis_name="c", subcore_axis_name="s")


def _mm_body(a_ref, w_ref, o_ref):
    o_ref[...] = jnp.dot(a_ref[...].astype(w_ref.dtype), w_ref[...],
                         preferred_element_type=jnp.float32
                         ).astype(o_ref.dtype)


def _matmul(a, w, bm, out_dtype=jnp.float32):
    rows, k = a.shape
    return pl.pallas_call(
        _mm_body,
        grid=(rows // bm,),
        in_specs=[pl.BlockSpec((bm, k), lambda i: (i, 0)),
                  pl.BlockSpec(w.shape, lambda i: (0, 0))],
        out_specs=pl.BlockSpec((bm, w.shape[1]), lambda i: (i, 0)),
        out_shape=jax.ShapeDtypeStruct((rows, w.shape[1]), out_dtype),
    )(a, w)


def _estep0_body(g_ref, a_ref, w_ref, msg_ref, hnew_ref):
    m = jnp.maximum(g_ref[...]
                    + a_ref[...].astype(jnp.float32), 0.0)
    msg_ref[...] = m
    hnew_ref[...] = jnp.dot(m.astype(jnp.bfloat16), w_ref[...],
                            preferred_element_type=jnp.float32
                            ).astype(jnp.bfloat16)


def _estep0(gb, ea2, whb):
    nb, hb = E // BM, EH // BM
    return pl.pallas_call(
        _estep0_body,
        grid=(nb,),
        in_specs=[pl.BlockSpec((BM, D), lambda i: (i, 0)),
                  pl.BlockSpec((BM, D), lambda i: (i % hb, 0)),
                  pl.BlockSpec((D, D), lambda i: (0, 0))],
        out_specs=[pl.BlockSpec((BM, D), lambda i: (i, 0)),
                   pl.BlockSpec((BM, D), lambda i: (i, 0))],
        out_shape=[jax.ShapeDtypeStruct((E, D), jnp.float32),
                   jax.ShapeDtypeStruct((E, D), jnp.bfloat16)],
    )(gb, ea2, whb)


def _estep_body(g_ref, a_ref, h_ref, w_ref, msg_ref, hnew_ref):
    m = jnp.maximum(g_ref[...]
                    + a_ref[...].astype(jnp.float32)
                    - h_ref[...].astype(jnp.float32), 0.0)
    msg_ref[...] = m
    hnew_ref[...] = jnp.dot(m.astype(jnp.bfloat16), w_ref[...],
                            preferred_element_type=jnp.float32
                            ).astype(jnp.bfloat16)


def _estep(gb, ea2, hmat, whb):
    nb, hb = E // BM, EH // BM
    return pl.pallas_call(
        _estep_body,
        grid=(nb,),
        in_specs=[pl.BlockSpec((BM, D), lambda i: (i, 0)),
                  pl.BlockSpec((BM, D), lambda i: (i % hb, 0)),
                  pl.BlockSpec((BM, D), lambda i: ((i + hb) % nb, 0)),
                  pl.BlockSpec((D, D), lambda i: (0, 0))],
        out_specs=[pl.BlockSpec((BM, D), lambda i: (i, 0)),
                   pl.BlockSpec((BM, D), lambda i: (i, 0))],
        out_shape=[jax.ShapeDtypeStruct((E, D), jnp.float32),
                   jax.ShapeDtypeStruct((E, D), jnp.bfloat16)],
    )(gb, ea2, hmat, whb)


def _edge_body(g_ref, a_ref, h_ref, o_ref):
    o_ref[...] = jnp.maximum(g_ref[...]
                             + a_ref[...].astype(jnp.float32)
                             - h_ref[...].astype(jnp.float32), 0.0)


def _edge_combine(gb, ea2, hmat):
    nb, hb = E // BM, EH // BM
    return pl.pallas_call(
        _edge_body,
        grid=(nb,),
        in_specs=[pl.BlockSpec((BM, D), lambda i: (i, 0)),
                  pl.BlockSpec((BM, D), lambda i: (i % hb, 0)),
                  pl.BlockSpec((BM, D), lambda i: ((i + hb) % nb, 0))],
        out_specs=pl.BlockSpec((BM, D), lambda i: (i, 0)),
        out_shape=jax.ShapeDtypeStruct((E, D), jnp.float32),
    )(gb, ea2, hmat)


def _tcomb_body(xh_ref, p0_ref, p1_ref, w_ref, o_ref):
    s = p0_ref[...] + p1_ref[...]
    o_ref[...] = xh_ref[...] + jnp.dot(s, w_ref[...],
                                       preferred_element_type=jnp.float32)


def _tcombine(xh, p0, p1, w):
    return pl.pallas_call(
        _tcomb_body,
        grid=(N // BN,),
        in_specs=[pl.BlockSpec((BN, D), lambda i: (i, 0)),
                  pl.BlockSpec((BN, D), lambda i: (i, 0)),
                  pl.BlockSpec((BN, D), lambda i: (i, 0)),
                  pl.BlockSpec((D, D), lambda i: (0, 0))],
        out_specs=pl.BlockSpec((BN, D), lambda i: (i, 0)),
        out_shape=jax.ShapeDtypeStruct((N, D), jnp.float32),
    )(xh, p0, p1, w)


def _final_body(x_ref, p0_ref, p1_ref, w1_ref, w2_ref, b_ref, o_ref):
    i = pl.program_id(0)
    m = p0_ref[...] + p1_ref[...]
    h = jnp.dot(x_ref[...], w1_ref[...], preferred_element_type=jnp.float32)
    h = h + jnp.dot(m, w2_ref[...], preferred_element_type=jnp.float32)
    h = jnp.maximum(h + b_ref[...], 0.0)
    s = jnp.sum(h, axis=0, keepdims=True) * (1.0 / N)

    @pl.when(i == 0)
    def _():
        o_ref[...] = jnp.zeros_like(o_ref)

    o_ref[...] += s


def _final(x, p0, p1, w1, w2, b):
    return pl.pallas_call(
        _final_body,
        grid=(N // BN,),
        in_specs=[pl.BlockSpec((BN, D), lambda i: (i, 0)),
                  pl.BlockSpec((BN, D), lambda i: (i, 0)),
                  pl.BlockSpec((BN, D), lambda i: (i, 0)),
                  pl.BlockSpec((D, D), lambda i: (0, 0)),
                  pl.BlockSpec((D, D), lambda i: (0, 0)),
                  pl.BlockSpec((1, D), lambda i: (0, 0))],
        out_specs=pl.BlockSpec((1, D), lambda i: (0, 0)),
        out_shape=jax.ShapeDtypeStruct((1, D), jnp.float32),
    )(x, p0, p1, w1, w2, b)


def _gather_sc_body(tab_ref, idx_hbm, out_ref, idx_v, buf0, buf1,
                    gsem0, gsem1, wsem0, wsem1):
    c = lax.axis_index("c")
    s = lax.axis_index("s")
    wid = c * NS + s
    e0 = wid * EPW
    pltpu.sync_copy(idx_hbm.at[wid], idx_v)

    bufs = (buf0, buf1)
    gsems = (gsem0, gsem1)
    wsems = (wsem0, wsem1)
    wdesc = [None, None]
    for g in range(NG):
        b = g & 1
        if wdesc[b] is not None:
            wdesc[b].wait()
        gds = [
            pltpu.async_copy(tab_ref.at[idx_v.at[g * NB + j]],
                             bufs[b].at[pl.ds(j * GC, GC)], gsems[b])
            for j in range(NB)
        ]
        for dsc in gds:
            dsc.wait()
        wdesc[b] = pltpu.async_copy(
            bufs[b], out_ref.at[pl.ds(e0 + g * GR, GR)], wsems[b])
    wdesc[0].wait()
    wdesc[1].wait()


@functools.partial(
    pl.kernel,
    out_type=jax.ShapeDtypeStruct((E, D), jnp.float32),
    mesh=_MESH,
    scratch_types=[
        pltpu.VMEM((CPW, GC), jnp.int32),
        pltpu.VMEM((GR, D), jnp.float32),
        pltpu.VMEM((GR, D), jnp.float32),
        pltpu.SemaphoreType.DMA,
        pltpu.SemaphoreType.DMA,
        pltpu.SemaphoreType.DMA,
        pltpu.SemaphoreType.DMA,
    ],
)
def _gather(tab_ref, idx_hbm, out_ref, idx_v, buf0, buf1,
            gsem0, gsem1, wsem0, wsem1):
    _gather_sc_body(tab_ref, idx_hbm, out_ref, idx_v, buf0, buf1,
                    gsem0, gsem1, wsem0, wsem1)


def _segsum_sc_body(msg_ref, idx_hbm, out_ref, idx_v, mb0, mb1, mb2, acc,
                    lsem0, lsem1, lsem2, ssem0, ssem1, ssem2):
    c = lax.axis_index("c")
    s = lax.axis_index("s")
    wid = c * NS + s
    e0 = wid * EPW
    pltpu.sync_copy(idx_hbm.at[wid], idx_v)

    zero = jnp.zeros((16,), jnp.float32)

    def zrow(i, _):
        for j in range(D // 16):
            mb0[i, pl.ds(j * 16, 16)] = zero
        return 0

    lax.fori_loop(0, ZC, zrow, 0)
    for z in range(NZC // NS + 1):
        ci = s + NS * z

        @pl.when(ci < NZC)
        def _():
            pltpu.sync_copy(mb0, acc.at[pl.ds(ci * ZC, ZC)])

    plsc.subcore_barrier()

    mbs = (mb0, mb1, mb2)
    lsems = (lsem0, lsem1, lsem2)
    ssems = (ssem0, ssem1, ssem2)
    ldesc = [None, None, None]
    sdesc = [None, None, None]
    for k in range(CPW + 1):
        if k < CPW:
            b = k % 3
            if sdesc[b] is not None:
                sdesc[b].wait()
            ldesc[b] = pltpu.async_copy(
                msg_ref.at[pl.ds(e0 + k * GC, GC)], mbs[b], lsems[b])
        if k > 0:
            pb = (k - 1) % 3
            ldesc[pb].wait()
            sdesc[pb] = pltpu.async_copy(
                mbs[pb], acc.at[idx_v.at[k - 1]], ssems[pb], add=True)
    for b in range(3):
        if sdesc[b] is not None:
            sdesc[b].wait()
    plsc.subcore_barrier()
    for z in range(NZC // NS + 1):
        ci = s + NS * z

        @pl.when(ci < NZC)
        def _():
            pltpu.sync_copy(acc.at[pl.ds(ci * ZC, ZC)],
                            out_ref.at[c, pl.ds(ci * ZC, ZC)])


@functools.partial(
    pl.kernel,
    out_type=jax.ShapeDtypeStruct((NC, N, D), jnp.float32),
    mesh=_MESH,
    scratch_types=[
        pltpu.VMEM((CPW, GC), jnp.int32),
        pltpu.VMEM((GC, D), jnp.float32),
        pltpu.VMEM((GC, D), jnp.float32),
        pltpu.VMEM((GC, D), jnp.float32),
        pltpu.VMEM_SHARED((N, D), jnp.float32),
        pltpu.SemaphoreType.DMA,
        pltpu.SemaphoreType.DMA,
        pltpu.SemaphoreType.DMA,
        pltpu.SemaphoreType.DMA,
        pltpu.SemaphoreType.DMA,
        pltpu.SemaphoreType.DMA,
    ],
)
def _segsum(msg_ref, idx_hbm, out_ref, idx_v, mb0, mb1, mb2, acc,
            lsem0, lsem1, lsem2, ssem0, ssem1, ssem2):
    _segsum_sc_body(msg_ref, idx_hbm, out_ref, idx_v, mb0, mb1, mb2, acc,
                    lsem0, lsem1, lsem2, ssem0, ssem1, ssem2)


def kernel(x, edge_attr, W_msg_i, W_h, W_o, b_o, edge_index):
    src = edge_index[0].astype(jnp.int32)
    dst = edge_index[1].astype(jnp.int32)
    srcr = src.reshape(NW, CPW, GC)
    dstr = dst.reshape(NW, CPW, GC)
    Wx = W_msg_i[:D]
    We = W_msg_i[D:]

    xh = _matmul(x, Wx, BN)
    ea2 = _matmul(edge_attr[:EH], We, BM,
                  out_dtype=jnp.bfloat16)
    whb = W_h.astype(jnp.bfloat16)

    g0 = _gather(xh, srcr)
    msg, hmat = _estep0(g0, ea2, whb)

    for _ in range(DEPTH - 2):
        parts = _segsum(msg, dstr)
        tb = _tcombine(xh, parts[0], parts[1], W_h)
        gb = _gather(tb, srcr)
        msg, hmat = _estep(gb, ea2, hmat, whb)

    parts = _segsum(msg, dstr)
    tb = _tcombine(xh, parts[0], parts[1], W_h)
    gb = _gather(tb, srcr)
    msg = _edge_combine(gb, ea2, hmat)

    parts = _segsum(msg, dstr)
    return _final(x, parts[0], parts[1], W_o[:D], W_o[D:],
                  b_o.reshape(1, D))
